# Initial kernel scaffold; baseline (speedup 1.0000x reference)
import jax
import jax.numpy as jnp
from jax.experimental import pallas as pl


def kernel(pcd_gad, feat, pcd_feadb_gad, feat_feadb, pos_w1, pos_b1, pos_g1, pos_be1, pos_w2, pos_b2, attn_w1, attn_b1, attn_g1, attn_be1, attn_w2, attn_b2):
    raise NotImplementedError("write your pallas kernel here")


# speedup vs baseline: 6.3075x; 6.3075x over previous
import functools

import jax
import jax.numpy as jnp
from jax import lax
from jax.experimental import pallas as pl
from jax.experimental.pallas import tpu as pltpu
from jax.experimental.pallas import tpu_sc as plsc

_HIGH = lax.Precision.HIGHEST
_BIG = 3.0e38


def _dot_bf16(a, b):
    return lax.dot_general(
        a.astype(jnp.bfloat16), b.astype(jnp.bfloat16),
        (((1,), (0,)), ((), ())), preferred_element_type=jnp.float32)


def _knn_body(nknn, m_total, q_ref, f_ref, o_ref):
    q = q_ref[0]
    f = f_ref[0]
    qs = jnp.sum(q * q, axis=1, keepdims=True)
    fs = jnp.sum(f * f, axis=0, keepdims=True)
    d = (-2.0 * _dot_bf16(q, f) + qs) + fs
    iot = lax.broadcasted_iota(jnp.int32, d.shape, 1)
    for j in range(nknn):
        mn = jnp.min(d, axis=1, keepdims=True)
        cand = jnp.where(d == mn, iot, m_total)
        ij = jnp.min(cand, axis=1)
        o_ref[0, j, :] = ij
        d = jnp.where(iot == ij[:, None], _BIG, d)


def _knn(q, f_t, nknn, tile_n, interpret=False):
    b, n, _ = q.shape
    m_total = f_t.shape[2]
    return pl.pallas_call(
        functools.partial(_knn_body, nknn, m_total),
        grid=(b, n // tile_n),
        in_specs=[
            pl.BlockSpec((1, tile_n, 8), lambda bb, i: (bb, i, 0)),
            pl.BlockSpec((1, 8, m_total), lambda bb, i: (bb, 0, 0)),
        ],
        out_specs=pl.BlockSpec((1, nknn, tile_n), lambda bb, i: (bb, 0, i)),
        out_shape=jax.ShapeDtypeStruct((b, nknn, n), jnp.int32),
        interpret=interpret,
    )(q, f_t)


def _sc_gather(tab_gad, tab_feat, idx_flat):
    p = idx_flat.shape[0]
    try:
        info = plsc.get_sparse_core_info()
        nc, ns = info.num_cores, info.num_subcores
    except Exception:
        nc, ns = 2, 16
    nw = nc * ns
    ch = 128
    per_w = p // nw
    n_ch = per_w // ch
    mesh = plsc.VectorSubcoreMesh(core_axis_name="c", subcore_axis_name="s")

    @functools.partial(
        pl.kernel, mesh=mesh,
        compiler_params=pltpu.CompilerParams(use_tc_tiling_on_sc=False),
        out_type=(jax.ShapeDtypeStruct((p, 16), jnp.float32),
                  jax.ShapeDtypeStruct((p, 64), jnp.float32)),
        scratch_types=[
            pltpu.VMEM((ch,), jnp.int32),
            pltpu.VMEM((ch, 16), jnp.float32),
            pltpu.VMEM((ch, 64), jnp.float32),
            pltpu.SemaphoreType.DMA,
            pltpu.SemaphoreType.DMA,
        ],
    )
    def k(tg_hbm, tf_hbm, idx_hbm, og_hbm, of_hbm, idx_v, gbuf, fbuf, s1, s2):
        wid = lax.axis_index("s") * nc + lax.axis_index("c")
        base = wid * per_w

        def body(c, carry):
            ofs = base + c * ch
            pltpu.sync_copy(idx_hbm.at[pl.ds(ofs, ch)], idx_v)
            cg = pltpu.async_copy(tg_hbm.at[idx_v], gbuf, s1)
            cf = pltpu.async_copy(tf_hbm.at[idx_v], fbuf, s2)
            cg.wait()
            cf.wait()
            pltpu.sync_copy(gbuf, og_hbm.at[pl.ds(ofs, ch)])
            pltpu.sync_copy(fbuf, of_hbm.at[pl.ds(ofs, ch)])
            return carry

        lax.fori_loop(0, n_ch, body, 0)

    return k(tab_gad, tab_feat, idx_flat)


def _m1_body(kpern, g_ref, k_ref, o_ref):
    i = pl.program_id(0)
    kb = k_ref[...]
    gb = g_ref[...]
    tnp = kb.shape[0]
    kr = jnp.broadcast_to(
        kb.reshape(tnp, 1, 16), (tnp, kpern, 16)).reshape(tnp * kpern, 16)
    x = kr - gb
    lane = lax.broadcasted_iota(jnp.int32, x.shape, 1)
    xa = jnp.where(lane == 14, 1.0, x)
    mm = lax.dot_general(xa, xa, (((0,), (0,)), ((), ())),
                         preferred_element_type=jnp.float32, precision=_HIGH)

    @pl.when(i == 0)
    def _():
        o_ref[...] = mm

    @pl.when(i > 0)
    def _():
        o_ref[...] += mm


def _moments1(ggad, keyg, kpern, tile_p, interpret=False):
    p = ggad.shape[0]
    tnp = tile_p // kpern
    return pl.pallas_call(
        functools.partial(_m1_body, kpern),
        grid=(p // tile_p,),
        in_specs=[
            pl.BlockSpec((tile_p, 16), lambda i: (i, 0)),
            pl.BlockSpec((tnp, 16), lambda i: (i, 0)),
        ],
        out_specs=pl.BlockSpec((16, 16), lambda i: (0, 0)),
        out_shape=jax.ShapeDtypeStruct((16, 16), jnp.float32),
        interpret=interpret,
    )(ggad, keyg)


def _pos_mlp(gg, gf, kg, kf, w1_ref, w2_ref, s1_ref, t1_ref, b2_ref, kpern):
    tnp = kg.shape[0]
    kgr = jnp.broadcast_to(
        kg.reshape(tnp, 1, 16), (tnp, kpern, 16)).reshape(tnp * kpern, 16)
    kfr = jnp.broadcast_to(
        kf.reshape(tnp, 1, 64), (tnp, kpern, 64)).reshape(tnp * kpern, 64)
    x = kgr - gg
    h = lax.dot_general(x, w1_ref[...], (((1,), (0,)), ((), ())),
                        preferred_element_type=jnp.float32, precision=_HIGH)
    pe = jnp.maximum(h * s1_ref[...] + t1_ref[...], 0.0)
    pemb = lax.dot_general(pe, w2_ref[...], (((1,), (0,)), ((), ())),
                           preferred_element_type=jnp.float32,
                           precision=_HIGH) + b2_ref[...]
    y = (kfr - gf) + pemb
    return y, pemb


def _ym_body(kpern, g_ref, f_ref, kg_ref, kf_ref, w1_ref, w2_ref,
             s1_ref, t1_ref, b2_ref, sy_ref, syy_ref):
    i = pl.program_id(0)
    y, _ = _pos_mlp(g_ref[...], f_ref[...], kg_ref[...], kf_ref[...],
                    w1_ref, w2_ref, s1_ref, t1_ref, b2_ref, kpern)
    sy = jnp.sum(y, axis=0, keepdims=True)
    yy = lax.dot_general(y, y, (((0,), (0,)), ((), ())),
                         preferred_element_type=jnp.float32, precision=_HIGH)

    @pl.when(i == 0)
    def _():
        sy_ref[...] = sy
        syy_ref[...] = yy

    @pl.when(i > 0)
    def _():
        sy_ref[...] += sy
        syy_ref[...] += yy


def _ymoments(ggad, gfeat, keyg, keyf, w1t, w2t, s1, t1, b2, kpern, tile_p,
              interpret=False):
    p = ggad.shape[0]
    tnp = tile_p // kpern
    vec = lambda: pl.BlockSpec((1, 64), lambda i: (0, 0))
    return pl.pallas_call(
        functools.partial(_ym_body, kpern),
        grid=(p // tile_p,),
        in_specs=[
            pl.BlockSpec((tile_p, 16), lambda i: (i, 0)),
            pl.BlockSpec((tile_p, 64), lambda i: (i, 0)),
            pl.BlockSpec((tnp, 16), lambda i: (i, 0)),
            pl.BlockSpec((tnp, 64), lambda i: (i, 0)),
            pl.BlockSpec((16, 64), lambda i: (0, 0)),
            pl.BlockSpec((64, 64), lambda i: (0, 0)),
            vec(), vec(), vec(),
        ],
        out_specs=(pl.BlockSpec((1, 64), lambda i: (0, 0)),
                   pl.BlockSpec((64, 64), lambda i: (0, 0))),
        out_shape=(jax.ShapeDtypeStruct((1, 64), jnp.float32),
                   jax.ShapeDtypeStruct((64, 64), jnp.float32)),
        interpret=interpret,
    )(ggad, gfeat, keyg, keyf, w1t, w2t, s1, t1, b2)


def _fin_body(kpern, g_ref, f_ref, kg_ref, kf_ref, w1_ref, w2_ref,
              s1_ref, t1_ref, b2_ref, w1a_ref, w2a_ref,
              s2_ref, t2_ref, b2a_ref, o_ref):
    gf = f_ref[...]
    y, pemb = _pos_mlp(g_ref[...], gf, kg_ref[...], kf_ref[...],
                       w1_ref, w2_ref, s1_ref, t1_ref, b2_ref, kpern)
    h1 = lax.dot_general(y, w1a_ref[...], (((1,), (0,)), ((), ())),
                         preferred_element_type=jnp.float32, precision=_HIGH)
    hh = jnp.maximum(h1 * s2_ref[...] + t2_ref[...], 0.0)
    s = lax.dot_general(hh, w2a_ref[...], (((1,), (0,)), ((), ())),
                        preferred_element_type=jnp.float32,
                        precision=_HIGH) + b2a_ref[...]
    tnp = s.shape[0] // kpern
    s3 = s.reshape(tnp, kpern, 64)
    mx = jnp.max(s3, axis=1, keepdims=True)
    e = jnp.exp(s3 - mx)
    ssum = jnp.sum(e, axis=1)
    g3 = (gf + pemb).reshape(tnp, kpern, 64)
    o_ref[...] = jnp.sum(e * g3, axis=1) / ssum


def _final(ggad, gfeat, keyg, keyf, w1t, w2t, s1, t1, b2,
           w1at, w2at, s2, t2, b2a, kpern, tile_p, interpret=False):
    p = ggad.shape[0]
    tnp = tile_p // kpern
    vec64 = lambda: pl.BlockSpec((1, 64), lambda i: (0, 0))
    vec256 = lambda: pl.BlockSpec((1, 256), lambda i: (0, 0))
    return pl.pallas_call(
        functools.partial(_fin_body, kpern),
        grid=(p // tile_p,),
        in_specs=[
            pl.BlockSpec((tile_p, 16), lambda i: (i, 0)),
            pl.BlockSpec((tile_p, 64), lambda i: (i, 0)),
            pl.BlockSpec((tnp, 16), lambda i: (i, 0)),
            pl.BlockSpec((tnp, 64), lambda i: (i, 0)),
            pl.BlockSpec((16, 64), lambda i: (0, 0)),
            pl.BlockSpec((64, 64), lambda i: (0, 0)),
            vec64(), vec64(), vec64(),
            pl.BlockSpec((64, 256), lambda i: (0, 0)),
            pl.BlockSpec((256, 64), lambda i: (0, 0)),
            vec256(), vec256(), vec64(),
        ],
        out_specs=pl.BlockSpec((tnp, 64), lambda i: (i, 0)),
        out_shape=jax.ShapeDtypeStruct((p // kpern, 64), jnp.float32),
        interpret=interpret,
    )(ggad, gfeat, keyg, keyf, w1t, w2t, s1, t1, b2,
      w1at, w2at, s2, t2, b2a)


def _pipeline(pcd_gad, feat, pcd_feadb_gad, feat_feadb,
              pos_w1, pos_b1, pos_g1, pos_be1, pos_w2, pos_b2,
              attn_w1, attn_b1, attn_g1, attn_be1, attn_w2, attn_b2,
              gather_fn, interpret=False):
    b, _, n = pcd_gad.shape
    m2 = pcd_feadb_gad.shape[2]
    m_total = n + m2
    nknn = 16
    tile_n = 512
    tile_p = 2048
    eps = 1e-5

    f32 = jnp.float32
    fus_gad = jnp.concatenate([pcd_gad, pcd_feadb_gad], axis=2)
    fus_feat = jnp.concatenate([feat, feat_feadb], axis=2)
    zpad5 = jnp.zeros((b, 5, m_total), f32)
    f_t = jnp.concatenate([fus_gad[:, 0:3, :], zpad5], axis=1)
    q = jnp.concatenate(
        [jnp.transpose(pcd_gad[:, 0:3, :], (0, 2, 1)),
         jnp.zeros((b, n, 5), f32)], axis=2)

    idx_t = _knn(q, f_t, nknn, tile_n, interpret=interpret)
    idx = jnp.transpose(idx_t, (0, 2, 1))
    idx_flat = (idx + (jnp.arange(b, dtype=jnp.int32) * m_total)[:, None, None]
                ).reshape(-1)

    tab_gad = jnp.concatenate(
        [jnp.transpose(fus_gad, (0, 2, 1)),
         jnp.zeros((b, m_total, 2), f32)], axis=2).reshape(b * m_total, 16)
    tab_feat = jnp.transpose(fus_feat, (0, 2, 1)).reshape(b * m_total, 64)

    ggad, gfeat = gather_fn(tab_gad, tab_feat, idx_flat)

    keyg = jnp.concatenate(
        [jnp.transpose(pcd_gad, (0, 2, 1)),
         jnp.zeros((b, n, 2), f32)], axis=2).reshape(b * n, 16)
    keyf = jnp.transpose(feat, (0, 2, 1)).reshape(b * n, 64)

    cnt = float(b * n * nknn)
    mom1 = _moments1(ggad, keyg, nknn, tile_p, interpret=interpret)
    sx = mom1[0:14, 14]
    sxx = mom1[0:14, 0:14]
    mu_x = sx / cnt
    cov_x = sxx / cnt - jnp.outer(mu_x, mu_x)
    m1 = pos_w1 @ mu_x + pos_b1
    v1 = jnp.sum((pos_w1 @ cov_x) * pos_w1, axis=1)
    scale1 = pos_g1 / jnp.sqrt(v1 + eps)
    t1 = pos_be1 + scale1 * (pos_b1 - m1)

    w1t = jnp.concatenate(
        [pos_w1.T, jnp.zeros((2, 64), f32)], axis=0)
    w2t = pos_w2.T
    s1r, t1r, b2r = scale1[None, :], t1[None, :], pos_b2[None, :]

    sy, syy = _ymoments(ggad, gfeat, keyg, keyf, w1t, w2t, s1r, t1r, b2r,
                        nknn, tile_p, interpret=interpret)
    mu_y = sy[0] / cnt
    cov_y = syy / cnt - jnp.outer(mu_y, mu_y)
    m2s = attn_w1 @ mu_y + attn_b1
    v2 = jnp.sum((attn_w1 @ cov_y) * attn_w1, axis=1)
    scale2 = attn_g1 / jnp.sqrt(v2 + eps)
    t2 = attn_be1 + scale2 * (attn_b1 - m2s)

    w1at = attn_w1.T
    w2at = attn_w2.T
    outf = _final(ggad, gfeat, keyg, keyf, w1t, w2t, s1r, t1r, b2r,
                  w1at, w2at, scale2[None, :], t2[None, :], attn_b2[None, :],
                  nknn, tile_p, interpret=interpret)
    return jnp.transpose(outf.reshape(b, n, 64), (0, 2, 1))


def kernel(pcd_gad, feat, pcd_feadb_gad, feat_feadb,
           pos_w1, pos_b1, pos_g1, pos_be1, pos_w2, pos_b2,
           attn_w1, attn_b1, attn_g1, attn_be1, attn_w2, attn_b2):
    return _pipeline(pcd_gad, feat, pcd_feadb_gad, feat_feadb,
                     pos_w1, pos_b1, pos_g1, pos_be1, pos_w2, pos_b2,
                     attn_w1, attn_b1, attn_g1, attn_be1, attn_w2, attn_b2,
                     gather_fn=_sc_gather)


# speedup vs baseline: 6.5648x; 1.0408x over previous
import functools

import jax
import jax.numpy as jnp
from jax import lax
from jax.experimental import pallas as pl
from jax.experimental.pallas import tpu as pltpu
from jax.experimental.pallas import tpu_sc as plsc

_HIGH = lax.Precision.HIGHEST
_BIG = 3.0e38


def _dot_bf16(a, b):
    return lax.dot_general(
        a.astype(jnp.bfloat16), b.astype(jnp.bfloat16),
        (((1,), (0,)), ((), ())), preferred_element_type=jnp.float32)


def _knn_body(nknn, m_total, q_ref, f_ref, o_ref):
    q = q_ref[0]
    f = f_ref[0]
    qs = jnp.sum(q * q, axis=1, keepdims=True)
    fs = jnp.sum(f * f, axis=0, keepdims=True)
    d = (-2.0 * _dot_bf16(q, f) + qs) + fs
    iot = lax.broadcasted_iota(jnp.int32, d.shape, 1)
    for j in range(nknn):
        mn = jnp.min(d, axis=1, keepdims=True)
        cand = jnp.where(d == mn, iot, m_total)
        ij = jnp.min(cand, axis=1)
        o_ref[0, j, :] = ij
        d = jnp.where(iot == ij[:, None], _BIG, d)


def _knn(q, f_t, nknn, tile_n, interpret=False):
    b, n, _ = q.shape
    m_total = f_t.shape[2]
    return pl.pallas_call(
        functools.partial(_knn_body, nknn, m_total),
        grid=(b, n // tile_n),
        in_specs=[
            pl.BlockSpec((1, tile_n, 8), lambda bb, i: (bb, i, 0)),
            pl.BlockSpec((1, 8, m_total), lambda bb, i: (bb, 0, 0)),
        ],
        out_specs=pl.BlockSpec((1, nknn, tile_n), lambda bb, i: (bb, 0, i)),
        out_shape=jax.ShapeDtypeStruct((b, nknn, n), jnp.int32),
        interpret=interpret,
    )(q, f_t)


def _sc_gather(tab_gad, tab_feat, idx_flat):
    p = idx_flat.shape[0]
    try:
        info = plsc.get_sparse_core_info()
        nc, ns = info.num_cores, info.num_subcores
    except Exception:
        nc, ns = 2, 16
    nw = nc * ns
    ch = 128
    per_w = p // nw
    n_ch = per_w // ch
    mesh = plsc.VectorSubcoreMesh(core_axis_name="c", subcore_axis_name="s")

    @functools.partial(
        pl.kernel, mesh=mesh,
        compiler_params=pltpu.CompilerParams(use_tc_tiling_on_sc=False),
        out_type=(jax.ShapeDtypeStruct((p, 16), jnp.float32),
                  jax.ShapeDtypeStruct((p, 64), jnp.float32)),
        scratch_types=[
            pltpu.VMEM((ch,), jnp.int32),
            pltpu.VMEM((ch, 16), jnp.float32),
            pltpu.VMEM((ch, 64), jnp.float32),
            pltpu.SemaphoreType.DMA,
            pltpu.SemaphoreType.DMA,
        ],
    )
    def k(tg_hbm, tf_hbm, idx_hbm, og_hbm, of_hbm, idx_v, gbuf, fbuf, s1, s2):
        wid = lax.axis_index("s") * nc + lax.axis_index("c")
        base = wid * per_w

        def body(c, carry):
            ofs = base + c * ch
            pltpu.sync_copy(idx_hbm.at[pl.ds(ofs, ch)], idx_v)
            cg = pltpu.async_copy(tg_hbm.at[idx_v], gbuf, s1)
            cf = pltpu.async_copy(tf_hbm.at[idx_v], fbuf, s2)
            cg.wait()
            cf.wait()
            pltpu.sync_copy(gbuf, og_hbm.at[pl.ds(ofs, ch)])
            pltpu.sync_copy(fbuf, of_hbm.at[pl.ds(ofs, ch)])
            return carry

        lax.fori_loop(0, n_ch, body, 0)

    return k(tab_gad, tab_feat, idx_flat)


def _m1_body(kpern, g_ref, k_ref, o_ref):
    i = pl.program_id(0)
    kb = k_ref[...]
    gb = g_ref[...]
    tnp = kb.shape[0]
    kr = jnp.broadcast_to(
        kb.reshape(tnp, 1, 16), (tnp, kpern, 16)).reshape(tnp * kpern, 16)
    x = kr - gb
    lane = lax.broadcasted_iota(jnp.int32, x.shape, 1)
    xa = jnp.where(lane == 14, 1.0, x)
    mm = lax.dot_general(xa, xa, (((0,), (0,)), ((), ())),
                         preferred_element_type=jnp.float32, precision=_HIGH)

    @pl.when(i == 0)
    def _():
        o_ref[...] = mm

    @pl.when(i > 0)
    def _():
        o_ref[...] += mm


def _moments1(ggad, keyg, kpern, tile_p, interpret=False):
    p = ggad.shape[0]
    tnp = tile_p // kpern
    return pl.pallas_call(
        functools.partial(_m1_body, kpern),
        grid=(p // tile_p,),
        in_specs=[
            pl.BlockSpec((tile_p, 16), lambda i: (i, 0)),
            pl.BlockSpec((tnp, 16), lambda i: (i, 0)),
        ],
        out_specs=pl.BlockSpec((16, 16), lambda i: (0, 0)),
        out_shape=jax.ShapeDtypeStruct((16, 16), jnp.float32),
        interpret=interpret,
    )(ggad, keyg)


def _pos_mlp(gg, gf, kg, kf, w1_ref, w2_ref, s1_ref, t1_ref, b2_ref, kpern):
    tnp = kg.shape[0]
    kgr = jnp.broadcast_to(
        kg.reshape(tnp, 1, 16), (tnp, kpern, 16)).reshape(tnp * kpern, 16)
    kfr = jnp.broadcast_to(
        kf.reshape(tnp, 1, 64), (tnp, kpern, 64)).reshape(tnp * kpern, 64)
    x = kgr - gg
    h = lax.dot_general(x, w1_ref[...], (((1,), (0,)), ((), ())),
                        preferred_element_type=jnp.float32, precision=_HIGH)
    pe = jnp.maximum(h * s1_ref[...] + t1_ref[...], 0.0)
    pemb = lax.dot_general(pe, w2_ref[...], (((1,), (0,)), ((), ())),
                           preferred_element_type=jnp.float32,
                           precision=_HIGH) + b2_ref[...]
    y = (kfr - gf) + pemb
    return y, pemb


def _ym_body(kpern, g_ref, f_ref, kg_ref, kf_ref, w1_ref, w2_ref,
             s1_ref, t1_ref, b2_ref, sy_ref, syy_ref):
    i = pl.program_id(0)
    y, _ = _pos_mlp(g_ref[...], f_ref[...], kg_ref[...], kf_ref[...],
                    w1_ref, w2_ref, s1_ref, t1_ref, b2_ref, kpern)
    sy = jnp.sum(y, axis=0, keepdims=True)
    yy = lax.dot_general(y, y, (((0,), (0,)), ((), ())),
                         preferred_element_type=jnp.float32, precision=_HIGH)

    @pl.when(i == 0)
    def _():
        sy_ref[...] = sy
        syy_ref[...] = yy

    @pl.when(i > 0)
    def _():
        sy_ref[...] += sy
        syy_ref[...] += yy


def _ymoments(ggad, gfeat, keyg, keyf, w1t, w2t, s1, t1, b2, kpern, tile_p,
              interpret=False):
    p = ggad.shape[0]
    tnp = tile_p // kpern
    vec = lambda: pl.BlockSpec((1, 64), lambda i: (0, 0))
    return pl.pallas_call(
        functools.partial(_ym_body, kpern),
        grid=(p // tile_p,),
        in_specs=[
            pl.BlockSpec((tile_p, 16), lambda i: (i, 0)),
            pl.BlockSpec((tile_p, 64), lambda i: (i, 0)),
            pl.BlockSpec((tnp, 16), lambda i: (i, 0)),
            pl.BlockSpec((tnp, 64), lambda i: (i, 0)),
            pl.BlockSpec((16, 64), lambda i: (0, 0)),
            pl.BlockSpec((64, 64), lambda i: (0, 0)),
            vec(), vec(), vec(),
        ],
        out_specs=(pl.BlockSpec((1, 64), lambda i: (0, 0)),
                   pl.BlockSpec((64, 64), lambda i: (0, 0))),
        out_shape=(jax.ShapeDtypeStruct((1, 64), jnp.float32),
                   jax.ShapeDtypeStruct((64, 64), jnp.float32)),
        interpret=interpret,
    )(ggad, gfeat, keyg, keyf, w1t, w2t, s1, t1, b2)


def _fin_body(kpern, g_ref, f_ref, kg_ref, kf_ref, w1_ref, w2_ref,
              s1_ref, t1_ref, b2_ref, w1a_ref, w2a_ref,
              s2_ref, t2_ref, b2a_ref, o_ref):
    gf = f_ref[...]
    y, pemb = _pos_mlp(g_ref[...], gf, kg_ref[...], kf_ref[...],
                       w1_ref, w2_ref, s1_ref, t1_ref, b2_ref, kpern)
    h1 = lax.dot_general(y, w1a_ref[...], (((1,), (0,)), ((), ())),
                         preferred_element_type=jnp.float32, precision=_HIGH)
    hh = jnp.maximum(h1 * s2_ref[...] + t2_ref[...], 0.0)
    s = lax.dot_general(hh, w2a_ref[...], (((1,), (0,)), ((), ())),
                        preferred_element_type=jnp.float32,
                        precision=_HIGH) + b2a_ref[...]
    tnp = s.shape[0] // kpern
    s3 = s.reshape(tnp, kpern, 64)
    mx = jnp.max(s3, axis=1, keepdims=True)
    e = jnp.exp(s3 - mx)
    ssum = jnp.sum(e, axis=1)
    g3 = (gf + pemb).reshape(tnp, kpern, 64)
    o_ref[...] = jnp.sum(e * g3, axis=1) / ssum


def _final(ggad, gfeat, keyg, keyf, w1t, w2t, s1, t1, b2,
           w1at, w2at, s2, t2, b2a, kpern, tile_p, interpret=False):
    p = ggad.shape[0]
    tnp = tile_p // kpern
    vec64 = lambda: pl.BlockSpec((1, 64), lambda i: (0, 0))
    vec256 = lambda: pl.BlockSpec((1, 256), lambda i: (0, 0))
    return pl.pallas_call(
        functools.partial(_fin_body, kpern),
        grid=(p // tile_p,),
        in_specs=[
            pl.BlockSpec((tile_p, 16), lambda i: (i, 0)),
            pl.BlockSpec((tile_p, 64), lambda i: (i, 0)),
            pl.BlockSpec((tnp, 16), lambda i: (i, 0)),
            pl.BlockSpec((tnp, 64), lambda i: (i, 0)),
            pl.BlockSpec((16, 64), lambda i: (0, 0)),
            pl.BlockSpec((64, 64), lambda i: (0, 0)),
            vec64(), vec64(), vec64(),
            pl.BlockSpec((64, 256), lambda i: (0, 0)),
            pl.BlockSpec((256, 64), lambda i: (0, 0)),
            vec256(), vec256(), vec64(),
        ],
        out_specs=pl.BlockSpec((tnp, 64), lambda i: (i, 0)),
        out_shape=jax.ShapeDtypeStruct((p // kpern, 64), jnp.float32),
        interpret=interpret,
    )(ggad, gfeat, keyg, keyf, w1t, w2t, s1, t1, b2,
      w1at, w2at, s2, t2, b2a)


def _pipeline(pcd_gad, feat, pcd_feadb_gad, feat_feadb,
              pos_w1, pos_b1, pos_g1, pos_be1, pos_w2, pos_b2,
              attn_w1, attn_b1, attn_g1, attn_be1, attn_w2, attn_b2,
              gather_fn, interpret=False):
    b, _, n = pcd_gad.shape
    m2 = pcd_feadb_gad.shape[2]
    m_total = n + m2
    nknn = 16
    tile_n = 1024
    tile_p = 2048
    eps = 1e-5

    f32 = jnp.float32
    fus_gad = jnp.concatenate([pcd_gad, pcd_feadb_gad], axis=2)
    fus_feat = jnp.concatenate([feat, feat_feadb], axis=2)
    zpad5 = jnp.zeros((b, 5, m_total), f32)
    f_t = jnp.concatenate([fus_gad[:, 0:3, :], zpad5], axis=1)
    q = jnp.concatenate(
        [jnp.transpose(pcd_gad[:, 0:3, :], (0, 2, 1)),
         jnp.zeros((b, n, 5), f32)], axis=2)

    idx_t = _knn(q, f_t, nknn, tile_n, interpret=interpret)
    idx = jnp.transpose(idx_t, (0, 2, 1))
    idx_flat = (idx + (jnp.arange(b, dtype=jnp.int32) * m_total)[:, None, None]
                ).reshape(-1)

    tab_gad = jnp.concatenate(
        [jnp.transpose(fus_gad, (0, 2, 1)),
         jnp.zeros((b, m_total, 2), f32)], axis=2).reshape(b * m_total, 16)
    tab_feat = jnp.transpose(fus_feat, (0, 2, 1)).reshape(b * m_total, 64)

    ggad, gfeat = gather_fn(tab_gad, tab_feat, idx_flat)

    keyg = jnp.concatenate(
        [jnp.transpose(pcd_gad, (0, 2, 1)),
         jnp.zeros((b, n, 2), f32)], axis=2).reshape(b * n, 16)
    keyf = jnp.transpose(feat, (0, 2, 1)).reshape(b * n, 64)

    cnt = float(b * n * nknn)
    mom1 = _moments1(ggad, keyg, nknn, tile_p, interpret=interpret)
    sx = mom1[0:14, 14]
    sxx = mom1[0:14, 0:14]
    mu_x = sx / cnt
    cov_x = sxx / cnt - jnp.outer(mu_x, mu_x)
    m1 = pos_w1 @ mu_x + pos_b1
    v1 = jnp.sum((pos_w1 @ cov_x) * pos_w1, axis=1)
    scale1 = pos_g1 / jnp.sqrt(v1 + eps)
    t1 = pos_be1 + scale1 * (pos_b1 - m1)

    w1t = jnp.concatenate(
        [pos_w1.T, jnp.zeros((2, 64), f32)], axis=0)
    w2t = pos_w2.T
    s1r, t1r, b2r = scale1[None, :], t1[None, :], pos_b2[None, :]

    sy, syy = _ymoments(ggad, gfeat, keyg, keyf, w1t, w2t, s1r, t1r, b2r,
                        nknn, tile_p, interpret=interpret)
    mu_y = sy[0] / cnt
    cov_y = syy / cnt - jnp.outer(mu_y, mu_y)
    m2s = attn_w1 @ mu_y + attn_b1
    v2 = jnp.sum((attn_w1 @ cov_y) * attn_w1, axis=1)
    scale2 = attn_g1 / jnp.sqrt(v2 + eps)
    t2 = attn_be1 + scale2 * (attn_b1 - m2s)

    w1at = attn_w1.T
    w2at = attn_w2.T
    outf = _final(ggad, gfeat, keyg, keyf, w1t, w2t, s1r, t1r, b2r,
                  w1at, w2at, scale2[None, :], t2[None, :], attn_b2[None, :],
                  nknn, tile_p, interpret=interpret)
    return jnp.transpose(outf.reshape(b, n, 64), (0, 2, 1))


def kernel(pcd_gad, feat, pcd_feadb_gad, feat_feadb,
           pos_w1, pos_b1, pos_g1, pos_be1, pos_w2, pos_b2,
           attn_w1, attn_b1, attn_g1, attn_be1, attn_w2, attn_b2):
    return _pipeline(pcd_gad, feat, pcd_feadb_gad, feat_feadb,
                     pos_w1, pos_b1, pos_g1, pos_be1, pos_w2, pos_b2,
                     attn_w1, attn_b1, attn_g1, attn_be1, attn_w2, attn_b2,
                     gather_fn=_sc_gather)
